# baseline (speedup 1.0000x reference)
import functools

import jax
import jax.numpy as jnp
from jax import lax
from jax.experimental import pallas as pl
from jax.experimental.pallas import tpu as pltpu
from jax.experimental.pallas import tpu_sc as plsc

D = 16
G1PAD = 102400
BG = 512
B3 = 2048
GW = 128
NW = 32
N_BLOCK = 10000


def _sc_gather(table, idx, c):
    m = idx.shape[0]
    mesh = plsc.VectorSubcoreMesh(core_axis_name="c", subcore_axis_name="s")

    @functools.partial(
        pl.kernel, mesh=mesh,
        out_type=jax.ShapeDtypeStruct((m, c), table.dtype),
        compiler_params=pltpu.CompilerParams(use_tc_tiling_on_sc=False))
    def k(table_hbm, idx_hbm, out_hbm):
        def body(i_vmem, o_vmem):
            pltpu.sync_copy(table_hbm.at[i_vmem.at[0]], o_vmem)

        pltpu.emit_pipeline(
            body,
            grid=(m // GW,),
            in_specs=[pl.BlockSpec((1, GW), lambda i: (0, i))],
            out_specs=[pl.BlockSpec((GW, c), lambda i: (i, 0))],
            core_axis_name=("c", "s"),
            dimension_semantics=(pltpu.PARALLEL,),
        )(idx_hbm, out_hbm)

    return k(table, idx.reshape(1, m))


def _k2_body(xj_ref, ea_ref, w1ax_ref, w1ae_ref, b1a_ref, w1b_ref, b1b_ref,
             out_ref):
    xj = xj_ref[...].reshape(D * BG, 16)
    ea6 = ea_ref[...].reshape(D * BG, 6)
    h = jnp.dot(xj, w1ax_ref[...], preferred_element_type=jnp.float32)
    h = h + jnp.dot(ea6, w1ae_ref[...], preferred_element_type=jnp.float32)
    h = jax.nn.relu(h + b1a_ref[...])
    msg = jnp.dot(h, w1b_ref[...], preferred_element_type=jnp.float32)
    msg = jax.nn.relu(msg + b1b_ref[...])
    msg = msg * ea6[:, 5:6]
    acc = msg[0:BG]
    for j in range(1, D):
        acc = jnp.maximum(acc, msg[j * BG:(j + 1) * BG])
    out_ref[...] = acc


def _edge_partials(xj3, ea3, w1ax, w1ae6, b1a, w1b, b1b):
    full = lambda i: (0, 0)
    return pl.pallas_call(
        _k2_body,
        grid=(G1PAD // BG,),
        in_specs=[
            pl.BlockSpec((D, BG, 16), lambda i: (0, i, 0)),
            pl.BlockSpec((D, BG, 6), lambda i: (0, i, 0)),
            pl.BlockSpec((16, 64), full),
            pl.BlockSpec((6, 64), full),
            pl.BlockSpec((1, 64), full),
            pl.BlockSpec((64, 64), full),
            pl.BlockSpec((1, 64), full),
        ],
        out_specs=pl.BlockSpec((BG, 64), lambda i: (i, 0)),
        out_shape=jax.ShapeDtypeStruct((G1PAD, 64), jnp.float32),
    )(xj3, ea3, w1ax, w1ae6, b1a, w1b, b1b)


def _k3_body(p_ref, nid_ref, out_ref, cval_ref, cnid_ref):
    @pl.when(pl.program_id(0) == 0)
    def _():
        cval_ref[...] = jnp.zeros((1, 64), jnp.float32)
        cnid_ref[0] = -1

    v = p_ref[...]
    nid = nid_ref[...]
    k = 1
    while k < B3:
        vs = jnp.concatenate(
            [jnp.zeros((k, 64), jnp.float32), v[:-k]], axis=0)
        ns = jnp.concatenate(
            [jnp.full((k, 1), -1, jnp.int32), nid[:-k]], axis=0)
        v = jnp.where(ns == nid, jnp.maximum(v, vs), v)
        k *= 2
    cn = cnid_ref[0]
    v = jnp.where(nid == cn, jnp.maximum(v, cval_ref[...]), v)
    cval_ref[...] = v[B3 - 1:B3]
    cnid_ref[0] = nid_ref[B3 - 1, 0]
    out_ref[...] = v


def _segmented_scan(partials, nid2):
    return pl.pallas_call(
        _k3_body,
        grid=(G1PAD // B3,),
        in_specs=[
            pl.BlockSpec((B3, 64), lambda i: (i, 0)),
            pl.BlockSpec((B3, 1), lambda i: (i, 0)),
        ],
        out_specs=pl.BlockSpec((B3, 64), lambda i: (i, 0)),
        out_shape=jax.ShapeDtypeStruct((G1PAD, 64), jnp.float32),
        scratch_shapes=[
            pltpu.VMEM((1, 64), jnp.float32),
            pltpu.SMEM((1,), jnp.int32),
        ],
    )(partials, nid2)


def _k5_body(x_ref, aggr_ref, w2ax_ref, w2aa_ref, b2a_ref, w2b_ref, b2b_ref,
             out_ref):
    x = x_ref[...]
    h2 = jnp.dot(x, w2ax_ref[...], preferred_element_type=jnp.float32)
    h2 = h2 + jnp.dot(aggr_ref[...], w2aa_ref[...],
                      preferred_element_type=jnp.float32)
    h2 = jax.nn.relu(h2 + b2a_ref[...])
    comb_all = jnp.dot(h2, w2b_ref[...], preferred_element_type=jnp.float32)
    comb_all = comb_all + b2b_ref[...]
    links = comb_all[:, 0:1]
    comb = comb_all[:, 1:5]
    nor = jnp.sqrt(jnp.sum(comb * comb, axis=1, keepdims=True))
    comb = comb / jnp.maximum(jnp.ones_like(nor), nor)
    out_ref[...] = jnp.concatenate([links, comb, x[:, :3]], axis=1)


def _node_mlp(x, aggr_pad, w2ax, w2aa, b2a, w2b, b2b):
    n = x.shape[0]
    full = lambda i: (0, 0)
    return pl.pallas_call(
        _k5_body,
        grid=(n // N_BLOCK,),
        in_specs=[
            pl.BlockSpec((N_BLOCK, 8), lambda i: (i, 0)),
            pl.BlockSpec((N_BLOCK, 64), lambda i: (i, 0)),
            pl.BlockSpec((8, 32), full),
            pl.BlockSpec((64, 32), full),
            pl.BlockSpec((1, 32), full),
            pl.BlockSpec((32, 5), full),
            pl.BlockSpec((1, 5), full),
        ],
        out_specs=pl.BlockSpec((N_BLOCK, 8), lambda i: (i, 0)),
        out_shape=jax.ShapeDtypeStruct((n, 8), jnp.float32),
    )(x, aggr_pad, w2ax, w2aa, b2a, w2b, b2b)


def kernel(x, edge_index, edge_attr, W1a, b1a, W1b, b1b, W2a, b2a, W2b, b2b):
    n = x.shape[0]
    e = edge_index.shape[1]
    src = edge_index[0]
    dst = edge_index[1]
    m_slots = D * G1PAD

    perm = jnp.argsort(dst)
    dst_s = dst[perm]
    src_s = src[perm]
    ea_s = edge_attr[perm]

    deg = jnp.zeros((n,), jnp.int32).at[dst].add(1)
    node_start = jnp.cumsum(deg) - deg
    ngroups = (deg + (D - 1)) // D
    group_base = jnp.cumsum(ngroups) - ngroups
    g1 = group_base[-1] + ngroups[-1]

    mark = jnp.zeros((G1PAD,), jnp.int32).at[group_base].add(1, mode="drop")
    gids = jnp.arange(G1PAD, dtype=jnp.int32)
    n_of_g = jnp.cumsum(mark) - 1
    n_of_g = jnp.where(gids < g1, n_of_g, n)

    nclip = jnp.clip(n_of_g, 0, n - 1)
    estart_g = jnp.take(node_start, nclip)
    gbase_g = jnp.take(group_base, nclip)
    nend_g = estart_g + jnp.take(deg, nclip)
    goff = estart_g + (gids - gbase_g) * D

    eoff = goff[:, None] + jnp.arange(D, dtype=jnp.int32)[None, :]
    valid = (gids[:, None] < g1) & (eoff < nend_g[:, None])
    eoff_cm = eoff.T.reshape(-1)
    valid_cm = valid.T.reshape(-1)
    slot_ids = jnp.arange(m_slots, dtype=jnp.int32)
    eoff_spread = jnp.where(valid_cm, eoff_cm, slot_ids % e)

    tbl = jnp.concatenate(
        [src_s.astype(jnp.float32)[:, None], ea_s,
         jnp.zeros((e, 10), jnp.float32)], axis=1)
    slots = _sc_gather(tbl, eoff_spread, 16)
    validf = valid_cm.astype(jnp.float32)
    pad_src = jnp.where(valid_cm, slots[:, 0].astype(jnp.int32),
                        slot_ids % n)
    pad_ea6 = jnp.concatenate(
        [slots[:, 1:6] * validf[:, None], validf[:, None]], axis=1)
    pad_ea3 = pad_ea6.reshape(D, G1PAD, 6)

    last_group = jnp.where(deg > 0, group_base + ngroups - 1, G1PAD - 1)
    lg_pad = jnp.concatenate(
        [last_group,
         jnp.arange(n, 53248, dtype=jnp.int32) % G1PAD]).astype(jnp.int32)
    nid2 = n_of_g[:, None]

    w1ax = jnp.concatenate([W1a[:8], jnp.zeros((8, 64), jnp.float32)], 0)
    w1ae6 = jnp.concatenate([W1a[8:13], jnp.zeros((1, 64), jnp.float32)], 0)
    w2ax, w2aa = W2a[:8], W2a[8:]
    b1a2 = b1a.reshape(1, 64)
    b1b2 = b1b.reshape(1, 64)
    b2a2 = b2a.reshape(1, 32)
    b2b2 = b2b.reshape(1, 5)

    def layer(xc):
        x16 = jnp.concatenate([xc, jnp.zeros((n, 8), jnp.float32)], axis=1)
        xj = _sc_gather(x16, pad_src, 16)
        xj3 = xj.reshape(D, G1PAD, 16)
        partials = _edge_partials(xj3, pad_ea3, w1ax, w1ae6, b1a2, W1b,
                                  b1b2)
        scanned = _segmented_scan(partials, nid2)
        aggr_pad = _sc_gather(scanned, lg_pad, 64)
        return _node_mlp(xc, aggr_pad, w2ax, w2aa, b2a2, W2b, b2b2)

    for _ in range(8):
        x = layer(x)
    return x


# speedup vs baseline: 3.8612x; 3.8612x over previous
import functools

import jax
import jax.numpy as jnp
from jax import lax
from jax.experimental import pallas as pl
from jax.experimental.pallas import tpu as pltpu
from jax.experimental.pallas import tpu_sc as plsc

D = 16
G1PAD = 102400
BG = 512
B3 = 2048
GW = 128
NW = 32
N_BLOCK = 10000


def _sc_gather(table, idx, c):
    m = idx.shape[0]
    mesh = plsc.VectorSubcoreMesh(core_axis_name="c", subcore_axis_name="s")

    @functools.partial(
        pl.kernel, mesh=mesh,
        out_type=jax.ShapeDtypeStruct((m, c), table.dtype),
        compiler_params=pltpu.CompilerParams(use_tc_tiling_on_sc=False))
    def k(table_hbm, idx_hbm, out_hbm):
        def body(i_vmem, o_vmem):
            pltpu.sync_copy(table_hbm.at[i_vmem.at[0]], o_vmem)

        pltpu.emit_pipeline(
            body,
            grid=(m // GW,),
            in_specs=[pl.BlockSpec((1, GW), lambda i: (0, i))],
            out_specs=[pl.BlockSpec((GW, c), lambda i: (i, 0))],
            core_axis_name=("c", "s"),
            dimension_semantics=(pltpu.PARALLEL,),
        )(idx_hbm, out_hbm)

    return k(table, idx.reshape(1, m))


def _k2_body(xj_ref, ea_ref, w1ax_ref, w1ae_ref, b1a_ref, w1b_ref, b1b_ref,
             out_ref):
    xj = xj_ref[...].reshape(D * BG, 16)
    ea6 = ea_ref[...].reshape(D * BG, 6)
    h = jnp.dot(xj, w1ax_ref[...], preferred_element_type=jnp.float32)
    h = h + jnp.dot(ea6, w1ae_ref[...], preferred_element_type=jnp.float32)
    h = jax.nn.relu(h + b1a_ref[...])
    msg = jnp.dot(h, w1b_ref[...], preferred_element_type=jnp.float32)
    msg = jax.nn.relu(msg + b1b_ref[...])
    msg = msg * ea6[:, 5:6]
    acc = msg[0:BG]
    for j in range(1, D):
        acc = jnp.maximum(acc, msg[j * BG:(j + 1) * BG])
    out_ref[...] = acc


def _edge_partials(xj3, ea3, w1ax, w1ae6, b1a, w1b, b1b):
    full = lambda i: (0, 0)
    return pl.pallas_call(
        _k2_body,
        grid=(G1PAD // BG,),
        in_specs=[
            pl.BlockSpec((D, BG, 16), lambda i: (0, i, 0)),
            pl.BlockSpec((D, BG, 6), lambda i: (0, i, 0)),
            pl.BlockSpec((16, 64), full),
            pl.BlockSpec((6, 64), full),
            pl.BlockSpec((1, 64), full),
            pl.BlockSpec((64, 64), full),
            pl.BlockSpec((1, 64), full),
        ],
        out_specs=pl.BlockSpec((BG, 64), lambda i: (i, 0)),
        out_shape=jax.ShapeDtypeStruct((G1PAD, 64), jnp.float32),
    )(xj3, ea3, w1ax, w1ae6, b1a, w1b, b1b)


def _k3_body(p_ref, nid_ref, out_ref, cval_ref, cnid_ref):
    @pl.when(pl.program_id(0) == 0)
    def _():
        cval_ref[...] = jnp.zeros((1, 64), jnp.float32)
        cnid_ref[0] = -1

    v = p_ref[...]
    nid = nid_ref[...]
    k = 1
    while k < B3:
        vs = jnp.concatenate(
            [jnp.zeros((k, 64), jnp.float32), v[:-k]], axis=0)
        ns = jnp.concatenate(
            [jnp.full((k, 1), -1, jnp.int32), nid[:-k]], axis=0)
        v = jnp.where(ns == nid, jnp.maximum(v, vs), v)
        k *= 2
    cn = cnid_ref[0]
    v = jnp.where(nid == cn, jnp.maximum(v, cval_ref[...]), v)
    cval_ref[...] = v[B3 - 1:B3]
    cnid_ref[0] = nid_ref[B3 - 1, 0]
    out_ref[...] = v


def _segmented_scan(partials, nid2):
    return pl.pallas_call(
        _k3_body,
        grid=(G1PAD // B3,),
        in_specs=[
            pl.BlockSpec((B3, 64), lambda i: (i, 0)),
            pl.BlockSpec((B3, 1), lambda i: (i, 0)),
        ],
        out_specs=pl.BlockSpec((B3, 64), lambda i: (i, 0)),
        out_shape=jax.ShapeDtypeStruct((G1PAD, 64), jnp.float32),
        scratch_shapes=[
            pltpu.VMEM((1, 64), jnp.float32),
            pltpu.SMEM((1,), jnp.int32),
        ],
    )(partials, nid2)


def _k5_body(x_ref, aggr_ref, w2ax_ref, w2aa_ref, b2a_ref, w2b_ref, b2b_ref,
             out_ref):
    x = x_ref[...]
    h2 = jnp.dot(x, w2ax_ref[...], preferred_element_type=jnp.float32)
    h2 = h2 + jnp.dot(aggr_ref[...], w2aa_ref[...],
                      preferred_element_type=jnp.float32)
    h2 = jax.nn.relu(h2 + b2a_ref[...])
    comb_all = jnp.dot(h2, w2b_ref[...], preferred_element_type=jnp.float32)
    comb_all = comb_all + b2b_ref[...]
    links = comb_all[:, 0:1]
    comb = comb_all[:, 1:5]
    nor = jnp.sqrt(jnp.sum(comb * comb, axis=1, keepdims=True))
    comb = comb / jnp.maximum(jnp.ones_like(nor), nor)
    out_ref[...] = jnp.concatenate([links, comb, x[:, :3]], axis=1)


def _node_mlp(x, aggr_pad, w2ax, w2aa, b2a, w2b, b2b):
    n = x.shape[0]
    full = lambda i: (0, 0)
    return pl.pallas_call(
        _k5_body,
        grid=(n // N_BLOCK,),
        in_specs=[
            pl.BlockSpec((N_BLOCK, 8), lambda i: (i, 0)),
            pl.BlockSpec((N_BLOCK, 64), lambda i: (i, 0)),
            pl.BlockSpec((8, 32), full),
            pl.BlockSpec((64, 32), full),
            pl.BlockSpec((1, 32), full),
            pl.BlockSpec((32, 5), full),
            pl.BlockSpec((1, 5), full),
        ],
        out_specs=pl.BlockSpec((N_BLOCK, 8), lambda i: (i, 0)),
        out_shape=jax.ShapeDtypeStruct((n, 8), jnp.float32),
    )(x, aggr_pad, w2ax, w2aa, b2a, w2b, b2b)


def kernel(x, edge_index, edge_attr, W1a, b1a, W1b, b1b, W2a, b2a, W2b, b2b):
    n = x.shape[0]
    e = edge_index.shape[1]
    src = edge_index[0]
    dst = edge_index[1]
    m_slots = D * G1PAD

    perm = jnp.argsort(dst)

    deg = jnp.zeros((n,), jnp.int32).at[dst].add(1)
    ngroups = (deg + (D - 1)) // D
    group_base = jnp.cumsum(ngroups) - ngroups
    g1 = group_base[-1] + ngroups[-1]

    mark = jnp.zeros((G1PAD,), jnp.int32).at[group_base].add(1, mode="drop")
    gids = jnp.arange(G1PAD, dtype=jnp.int32)
    n_of_g = jnp.cumsum(mark) - 1
    n_of_g = jnp.where(gids < g1, n_of_g, n)

    nend_g = jnp.cumsum(
        jnp.zeros((G1PAD,), jnp.int32).at[group_base].add(deg, mode="drop"))
    deg_prev = jnp.concatenate([jnp.zeros((1,), jnp.int32), deg[:-1]])
    ng_prev = jnp.concatenate([jnp.zeros((1,), jnp.int32), ngroups[:-1]])
    w = jnp.full((G1PAD,), D, jnp.int32).at[group_base].add(
        deg_prev - D * ng_prev, mode="drop")
    goff = jnp.cumsum(w) - D

    eoff = goff[None, :] + jnp.arange(D, dtype=jnp.int32)[:, None]
    valid = (gids[None, :] < g1) & (eoff < nend_g[None, :])
    valid_cm = valid.reshape(-1)
    slot_ids = jnp.arange(m_slots, dtype=jnp.int32)
    eoff_spread = jnp.where(valid_cm, eoff.reshape(-1), slot_ids % e)

    tbl = jnp.concatenate(
        [src.astype(jnp.float32)[:, None], edge_attr,
         jnp.zeros((e, 10), jnp.float32)], axis=1)
    eperm_pad = jnp.concatenate(
        [perm.astype(jnp.int32),
         jnp.arange(e, 802816, dtype=jnp.int32) % e])
    tbl_s = _sc_gather(tbl, eperm_pad, 16)
    slots = _sc_gather(tbl_s, eoff_spread, 16)
    validf = valid_cm.astype(jnp.float32)
    pad_src = jnp.where(valid_cm, slots[:, 0].astype(jnp.int32),
                        slot_ids % n)
    pad_ea6 = jnp.concatenate(
        [slots[:, 1:6] * validf[:, None], validf[:, None]], axis=1)
    pad_ea3 = pad_ea6.reshape(D, G1PAD, 6)

    last_group = jnp.where(deg > 0, group_base + ngroups - 1, G1PAD - 1)
    lg_pad = jnp.concatenate(
        [last_group,
         jnp.arange(n, 53248, dtype=jnp.int32) % G1PAD]).astype(jnp.int32)
    nid2 = n_of_g[:, None]

    w1ax = jnp.concatenate([W1a[:8], jnp.zeros((8, 64), jnp.float32)], 0)
    w1ae6 = jnp.concatenate([W1a[8:13], jnp.zeros((1, 64), jnp.float32)], 0)
    w2ax, w2aa = W2a[:8], W2a[8:]
    b1a2 = b1a.reshape(1, 64)
    b1b2 = b1b.reshape(1, 64)
    b2a2 = b2a.reshape(1, 32)
    b2b2 = b2b.reshape(1, 5)

    def layer(xc):
        x16 = jnp.concatenate([xc, jnp.zeros((n, 8), jnp.float32)], axis=1)
        xj = _sc_gather(x16, pad_src, 16)
        xj3 = xj.reshape(D, G1PAD, 16)
        partials = _edge_partials(xj3, pad_ea3, w1ax, w1ae6, b1a2, W1b,
                                  b1b2)
        scanned = _segmented_scan(partials, nid2)
        aggr_pad = _sc_gather(scanned, lg_pad, 64)
        return _node_mlp(xc, aggr_pad, w2ax, w2aa, b2a2, W2b, b2b2)

    for _ in range(8):
        x = layer(x)
    return x


# speedup vs baseline: 4.0324x; 1.0443x over previous
import functools

import jax
import jax.numpy as jnp
from jax import lax
from jax.experimental import pallas as pl
from jax.experimental.pallas import tpu as pltpu
from jax.experimental.pallas import tpu_sc as plsc

D = 16
G1PAD = 102400
BG = 512
B3 = 2048
GW = 128
NW = 32
N_BLOCK = 10000


def _sc_gather(table, idx, c):
    m = idx.shape[0]
    mesh = plsc.VectorSubcoreMesh(core_axis_name="c", subcore_axis_name="s")

    @functools.partial(
        pl.kernel, mesh=mesh,
        out_type=jax.ShapeDtypeStruct((m, c), table.dtype),
        compiler_params=pltpu.CompilerParams(use_tc_tiling_on_sc=False))
    def k(table_hbm, idx_hbm, out_hbm):
        def body(i_vmem, o_vmem):
            pltpu.sync_copy(table_hbm.at[i_vmem.at[0]], o_vmem)

        pltpu.emit_pipeline(
            body,
            grid=(m // GW,),
            in_specs=[pl.BlockSpec((1, GW), lambda i: (0, i))],
            out_specs=[pl.BlockSpec((GW, c), lambda i: (i, 0))],
            core_axis_name=("c", "s"),
            dimension_semantics=(pltpu.PARALLEL,),
        )(idx_hbm, out_hbm)

    return k(table, idx.reshape(1, m))


def _k2_body(xj_ref, ea_ref, w1ax_ref, w1ae_ref, b1a_ref, w1b_ref, b1b_ref,
             out_ref):
    xj = xj_ref[...].reshape(D * BG, 16)
    ea6 = ea_ref[...].reshape(D * BG, 6)
    h = jnp.dot(xj, w1ax_ref[...], preferred_element_type=jnp.float32)
    h = h + jnp.dot(ea6, w1ae_ref[...], preferred_element_type=jnp.float32)
    h = jax.nn.relu(h + b1a_ref[...])
    msg = jnp.dot(h, w1b_ref[...], preferred_element_type=jnp.float32)
    msg = jax.nn.relu(msg + b1b_ref[...])
    msg = msg * ea6[:, 5:6]
    acc = msg[0:BG]
    for j in range(1, D):
        acc = jnp.maximum(acc, msg[j * BG:(j + 1) * BG])
    out_ref[...] = acc


def _edge_partials(xj3, ea3, w1ax, w1ae6, b1a, w1b, b1b, half):
    full = lambda i: (0, 0)
    hoff = half * (G1PAD // 2 // BG)
    return pl.pallas_call(
        _k2_body,
        grid=(G1PAD // 2 // BG,),
        in_specs=[
            pl.BlockSpec((D, BG, 16), lambda i: (0, i, 0)),
            pl.BlockSpec((D, BG, 6), lambda i: (0, i + hoff, 0)),
            pl.BlockSpec((16, 64), full),
            pl.BlockSpec((6, 64), full),
            pl.BlockSpec((1, 64), full),
            pl.BlockSpec((64, 64), full),
            pl.BlockSpec((1, 64), full),
        ],
        out_specs=pl.BlockSpec((BG, 64), lambda i: (i, 0)),
        out_shape=jax.ShapeDtypeStruct((G1PAD // 2, 64), jnp.float32),
    )(xj3, ea3, w1ax, w1ae6, b1a, w1b, b1b)


def _k3_body(p_ref, nid_ref, out_ref, cval_ref, cnid_ref):
    @pl.when(pl.program_id(0) == 0)
    def _():
        cval_ref[...] = jnp.zeros((1, 64), jnp.float32)
        cnid_ref[0] = -1

    v = p_ref[...]
    nid = nid_ref[...]
    k = 1
    while k < B3:
        vs = jnp.concatenate(
            [jnp.zeros((k, 64), jnp.float32), v[:-k]], axis=0)
        ns = jnp.concatenate(
            [jnp.full((k, 1), -1, jnp.int32), nid[:-k]], axis=0)
        v = jnp.where(ns == nid, jnp.maximum(v, vs), v)
        k *= 2
    cn = cnid_ref[0]
    v = jnp.where(nid == cn, jnp.maximum(v, cval_ref[...]), v)
    cval_ref[...] = v[B3 - 1:B3]
    cnid_ref[0] = nid_ref[B3 - 1, 0]
    out_ref[...] = v


def _segmented_scan(partials, nid2):
    return pl.pallas_call(
        _k3_body,
        grid=(G1PAD // B3,),
        in_specs=[
            pl.BlockSpec((B3, 64), lambda i: (i, 0)),
            pl.BlockSpec((B3, 1), lambda i: (i, 0)),
        ],
        out_specs=pl.BlockSpec((B3, 64), lambda i: (i, 0)),
        out_shape=jax.ShapeDtypeStruct((G1PAD, 64), jnp.float32),
        scratch_shapes=[
            pltpu.VMEM((1, 64), jnp.float32),
            pltpu.SMEM((1,), jnp.int32),
        ],
    )(partials, nid2)


def _k5_body(x_ref, aggr_ref, w2ax_ref, w2aa_ref, b2a_ref, w2b_ref, b2b_ref,
             out_ref):
    x = x_ref[...]
    h2 = jnp.dot(x, w2ax_ref[...], preferred_element_type=jnp.float32)
    h2 = h2 + jnp.dot(aggr_ref[...], w2aa_ref[...],
                      preferred_element_type=jnp.float32)
    h2 = jax.nn.relu(h2 + b2a_ref[...])
    comb_all = jnp.dot(h2, w2b_ref[...], preferred_element_type=jnp.float32)
    comb_all = comb_all + b2b_ref[...]
    links = comb_all[:, 0:1]
    comb = comb_all[:, 1:5]
    nor = jnp.sqrt(jnp.sum(comb * comb, axis=1, keepdims=True))
    comb = comb / jnp.maximum(jnp.ones_like(nor), nor)
    out_ref[...] = jnp.concatenate([links, comb, x[:, :3]], axis=1)


def _node_mlp(x, aggr_pad, w2ax, w2aa, b2a, w2b, b2b):
    n = x.shape[0]
    full = lambda i: (0, 0)
    return pl.pallas_call(
        _k5_body,
        grid=(n // N_BLOCK,),
        in_specs=[
            pl.BlockSpec((N_BLOCK, 8), lambda i: (i, 0)),
            pl.BlockSpec((N_BLOCK, 64), lambda i: (i, 0)),
            pl.BlockSpec((8, 32), full),
            pl.BlockSpec((64, 32), full),
            pl.BlockSpec((1, 32), full),
            pl.BlockSpec((32, 5), full),
            pl.BlockSpec((1, 5), full),
        ],
        out_specs=pl.BlockSpec((N_BLOCK, 8), lambda i: (i, 0)),
        out_shape=jax.ShapeDtypeStruct((n, 8), jnp.float32),
    )(x, aggr_pad, w2ax, w2aa, b2a, w2b, b2b)


def kernel(x, edge_index, edge_attr, W1a, b1a, W1b, b1b, W2a, b2a, W2b, b2b):
    n = x.shape[0]
    e = edge_index.shape[1]
    src = edge_index[0]
    dst = edge_index[1]
    m_slots = D * G1PAD

    perm = jnp.argsort(dst)

    deg = jnp.zeros((n,), jnp.int32).at[dst].add(1)
    ngroups = (deg + (D - 1)) // D
    group_base = jnp.cumsum(ngroups) - ngroups
    g1 = group_base[-1] + ngroups[-1]

    mark = jnp.zeros((G1PAD,), jnp.int32).at[group_base].add(1, mode="drop")
    gids = jnp.arange(G1PAD, dtype=jnp.int32)
    n_of_g = jnp.cumsum(mark) - 1
    n_of_g = jnp.where(gids < g1, n_of_g, n)

    nend_g = jnp.cumsum(
        jnp.zeros((G1PAD,), jnp.int32).at[group_base].add(deg, mode="drop"))
    deg_prev = jnp.concatenate([jnp.zeros((1,), jnp.int32), deg[:-1]])
    ng_prev = jnp.concatenate([jnp.zeros((1,), jnp.int32), ngroups[:-1]])
    w = jnp.full((G1PAD,), D, jnp.int32).at[group_base].add(
        deg_prev - D * ng_prev, mode="drop")
    goff = jnp.cumsum(w) - D

    eoff = goff[None, :] + jnp.arange(D, dtype=jnp.int32)[:, None]
    valid = (gids[None, :] < g1) & (eoff < nend_g[None, :])
    valid_cm = valid.reshape(-1)
    slot_ids = jnp.arange(m_slots, dtype=jnp.int32)
    eoff_spread = jnp.where(valid_cm, eoff.reshape(-1), slot_ids % e)

    tbl = jnp.concatenate(
        [src.astype(jnp.float32)[:, None], edge_attr,
         jnp.zeros((e, 10), jnp.float32)], axis=1)
    eperm_pad = jnp.concatenate(
        [perm.astype(jnp.int32),
         jnp.arange(e, 802816, dtype=jnp.int32) % e])
    tbl_s = _sc_gather(tbl, eperm_pad, 16)
    slots = _sc_gather(tbl_s, eoff_spread, 16)
    validf = valid_cm.astype(jnp.float32)
    pad_src = jnp.where(valid_cm, slots[:, 0].astype(jnp.int32),
                        slot_ids % n)
    pad_ea6 = jnp.concatenate(
        [slots[:, 1:6] * validf[:, None], validf[:, None]], axis=1)
    pad_ea3 = pad_ea6.reshape(D, G1PAD, 6)
    h = G1PAD // 2
    pad_src3 = pad_src.reshape(D, G1PAD)
    pad_src_a = pad_src3[:, :h].reshape(-1)
    pad_src_b = pad_src3[:, h:].reshape(-1)

    last_group = jnp.where(deg > 0, group_base + ngroups - 1, G1PAD - 1)
    lg_pad = jnp.concatenate(
        [last_group,
         jnp.arange(n, 53248, dtype=jnp.int32) % G1PAD]).astype(jnp.int32)
    nid2 = n_of_g[:, None]

    w1ax = jnp.concatenate([W1a[:8], jnp.zeros((8, 64), jnp.float32)], 0)
    w1ae6 = jnp.concatenate([W1a[8:13], jnp.zeros((1, 64), jnp.float32)], 0)
    w2ax, w2aa = W2a[:8], W2a[8:]
    b1a2 = b1a.reshape(1, 64)
    b1b2 = b1b.reshape(1, 64)
    b2a2 = b2a.reshape(1, 32)
    b2b2 = b2b.reshape(1, 5)

    def layer(xc):
        x16 = jnp.concatenate([xc, jnp.zeros((n, 8), jnp.float32)], axis=1)
        xj_a = _sc_gather(x16, pad_src_a, 16)
        p_a = _edge_partials(xj_a.reshape(D, h, 16), pad_ea3, w1ax, w1ae6,
                             b1a2, W1b, b1b2, 0)
        xj_b = _sc_gather(x16, pad_src_b, 16)
        p_b = _edge_partials(xj_b.reshape(D, h, 16), pad_ea3, w1ax, w1ae6,
                             b1a2, W1b, b1b2, 1)
        partials = jnp.concatenate([p_a, p_b], axis=0)
        scanned = _segmented_scan(partials, nid2)
        aggr_pad = _sc_gather(scanned, lg_pad, 64)
        return _node_mlp(xc, aggr_pad, w2ax, w2aa, b2a2, W2b, b2b2)

    for _ in range(8):
        x = layer(x)
    return x
